# Initial kernel scaffold; baseline (speedup 1.0000x reference)
import jax
import jax.numpy as jnp
from jax.experimental import pallas as pl


def kernel(input_spikes, input_weights, input_delays, thresholds):
    raise NotImplementedError("write your pallas kernel here")


# speedup vs baseline: 1.9237x; 1.9237x over previous
import functools

import jax
import jax.numpy as jnp
from jax.experimental import pallas as pl
from jax.experimental.pallas import tpu as pltpu


def _bitonic_sort_kv(t, w, n):
    idx = jax.lax.broadcasted_iota(jnp.int32, (n, 1), 0)
    k = 2
    while k <= n:
        d = k // 2
        while d >= 1:
            first = (idx & d) == 0
            bit_d = (idx & d) // d
            bit_k = (idx & k) // k
            f = (1 - 2 * (bit_d ^ bit_k)).astype(t.dtype)
            p_t = jnp.where(first, jnp.roll(t, -d, axis=0), jnp.roll(t, d, axis=0))
            p_w = jnp.where(first, jnp.roll(w, -d, axis=0), jnp.roll(w, d, axis=0))
            keep = (f * (t - p_t)) <= 0.0
            t = jnp.where(keep, t, p_t)
            w = jnp.where(keep, w, p_w)
            d //= 2
        k *= 2
    return t, w


def _cumsum0(x, n):
    idx = jax.lax.broadcasted_iota(jnp.int32, (n, 1), 0)
    s = 1
    while s < n:
        x = x + jnp.where(idx >= s, jnp.roll(x, s, axis=0), 0.0)
        s *= 2
    return x


def _eqtime_kernel(s_ref, d_ref, w_ref, th_ref, o_ref, *, n_pre):
    t = s_ref[0] + d_ref[...]
    w = w_ref[...]
    t, w = _bitonic_sort_kv(t, w, n_pre)

    cw = _cumsum0(w, n_pre)
    cwt = _cumsum0(w * t, n_pre)

    theta = th_ref[...]
    eps = 1e-10
    cw_ok = cw > eps
    tmp = (theta + cwt) / jnp.where(cw_ok, cw, 1.0)
    tmp = jnp.where(cw_ok, tmp, jnp.inf)

    idx = jax.lax.broadcasted_iota(jnp.int32, (n_pre, 1), 0)
    t_next = jnp.where(idx == n_pre - 1, jnp.inf, jnp.roll(t, -1, axis=0))
    valid = (tmp >= t) & (tmp <= t_next)
    cand = jnp.where(valid, tmp, jnp.inf)
    o_ref[0] = jnp.min(cand, axis=0, keepdims=True)


def kernel(input_spikes, input_weights, input_delays, thresholds):
    b, n_pre = input_spikes.shape
    n_post = input_weights.shape[1]
    tj = min(128, n_post)
    n_jt = n_post // tj

    s_t = input_spikes.reshape(b, n_pre, 1)
    th = thresholds.reshape(1, n_post)

    grid = (n_jt, b)
    out = pl.pallas_call(
        functools.partial(_eqtime_kernel, n_pre=n_pre),
        grid=grid,
        in_specs=[
            pl.BlockSpec((1, n_pre, 1), lambda jt, bb: (bb, 0, 0)),
            pl.BlockSpec((n_pre, tj), lambda jt, bb: (0, jt)),
            pl.BlockSpec((n_pre, tj), lambda jt, bb: (0, jt)),
            pl.BlockSpec((1, tj), lambda jt, bb: (0, jt)),
        ],
        out_specs=pl.BlockSpec((1, 1, tj), lambda jt, bb: (bb, 0, jt)),
        out_shape=jax.ShapeDtypeStruct((b, 1, n_post), jnp.float32),
        compiler_params=pltpu.CompilerParams(
            dimension_semantics=("parallel", "arbitrary"),
        ),
    )(s_t, input_delays, input_weights, th)
    return out.reshape(b, n_post)


# speedup vs baseline: 4.9921x; 2.5951x over previous
import functools

import jax
import jax.numpy as jnp
from jax.experimental import pallas as pl
from jax.experimental.pallas import tpu as pltpu

_S = 8


def _viota(axis_len, axis):
    shape = [1, 1, 1]
    shape[axis] = axis_len
    return jax.lax.broadcasted_iota(jnp.int32, tuple(shape), axis)


def _bitonic_sort_kv(t, w, n):
    _V = n // _S
    lanes = t.shape[2]
    s_iota = _viota(_S, 1)
    k = 2
    while k <= n:
        d = k // 2
        while d >= 1:
            if d <= _V // 2:
                g = _V // (2 * d)
                tv = t.reshape(g, 2, d, _S, lanes)
                wv = w.reshape(g, 2, d, _S, lanes)
                ta, tb = tv[:, 0], tv[:, 1]
                wa, wb = wv[:, 0], wv[:, 1]
                cmp = ta <= tb
                t_lo = jnp.minimum(ta, tb)
                t_hi = jnp.maximum(ta, tb)
                w_lo = jnp.where(cmp, wa, wb)
                w_hi = jnp.where(cmp, wb, wa)
                if k == n:
                    t0, t1, w0, w1 = t_lo, t_hi, w_lo, w_hi
                else:
                    if k < _V:
                        gidx = jax.lax.broadcasted_iota(
                            jnp.int32, (g, 1, 1, 1), 0)
                        asc = ((gidx * (2 * d)) & k) == 0
                    else:
                        asc = (s_iota.reshape(1, 1, _S, 1) & (k // _V)) == 0
                    t0 = jnp.where(asc, t_lo, t_hi)
                    t1 = jnp.where(asc, t_hi, t_lo)
                    w0 = jnp.where(asc, w_lo, w_hi)
                    w1 = jnp.where(asc, w_hi, w_lo)
                t = jnp.stack([t0, t1], axis=1).reshape(_V, _S, lanes)
                w = jnp.stack([w0, w1], axis=1).reshape(_V, _S, lanes)
            else:
                e = d // _V
                bit_d = (s_iota & e) // e
                if k == n:
                    bit_k = jnp.zeros_like(s_iota)
                else:
                    kk = k // _V
                    bit_k = (s_iota & kk) // kk
                first = bit_d == 0
                f = (1 - 2 * (bit_d ^ bit_k)).astype(t.dtype)
                p_t = jnp.where(first, jnp.roll(t, -e, axis=1),
                                jnp.roll(t, e, axis=1))
                p_w = jnp.where(first, jnp.roll(w, -e, axis=1),
                                jnp.roll(w, e, axis=1))
                keep = (f * (t - p_t)) <= 0.0
                t = jnp.where(keep, t, p_t)
                w = jnp.where(keep, w, p_w)
            d //= 2
        k *= 2
    return t, w


def _cumsum_logical(x):
    _V = x.shape[0]
    v_iota = _viota(_V, 0)
    s_iota = _viota(_S, 1)
    sh = 1
    while sh < _V:
        x = x + jnp.where(v_iota >= sh, jnp.roll(x, sh, axis=0), 0.0)
        sh *= 2
    tot = x[_V - 1:_V]
    sh = 1
    while sh < _S:
        tot = tot + jnp.where(s_iota >= sh, jnp.roll(tot, sh, axis=1), 0.0)
        sh *= 2
    excl = jnp.where(s_iota >= 1, jnp.roll(tot, 1, axis=1), 0.0)
    return x + excl


def _eqtime_kernel(s_ref, d_ref, w_ref, th_ref, o_ref, *, n_pre):
    _V = n_pre // _S
    lanes = d_ref.shape[1]
    t = (s_ref[0] + d_ref[...]).reshape(_V, _S, lanes)
    w = w_ref[...].reshape(_V, _S, lanes)
    t, w = _bitonic_sort_kv(t, w, n_pre)

    cw = _cumsum_logical(w)
    cwt = _cumsum_logical(w * t)

    theta = th_ref[...].reshape(1, 1, lanes)
    eps = 1e-10
    cw_ok = cw > eps
    tmp = (theta + cwt) / jnp.where(cw_ok, cw, 1.0)
    tmp = jnp.where(cw_ok, tmp, jnp.inf)

    v_iota = _viota(_V, 0)
    s_iota = _viota(_S, 1)
    nxt_in_run = jnp.roll(t, -1, axis=0)
    run_head = jnp.roll(jnp.roll(t, -1, axis=1), -1, axis=0)
    run_head = jnp.where(s_iota == _S - 1, jnp.inf, run_head)
    t_next = jnp.where(v_iota == _V - 1, run_head, nxt_in_run)

    valid = (tmp >= t) & (tmp <= t_next)
    cand = jnp.where(valid, tmp, jnp.inf)
    o_ref[0] = jnp.min(cand, axis=(0, 1)).reshape(1, lanes)


def kernel(input_spikes, input_weights, input_delays, thresholds):
    b, n_pre = input_spikes.shape
    n_post = input_weights.shape[1]
    tj = min(128, n_post)
    n_jt = n_post // tj

    s_t = input_spikes.reshape(b, n_pre, 1)
    th = thresholds.reshape(1, n_post)

    grid = (n_jt, b)
    out = pl.pallas_call(
        functools.partial(_eqtime_kernel, n_pre=n_pre),
        grid=grid,
        in_specs=[
            pl.BlockSpec((1, n_pre, 1), lambda jt, bb: (bb, 0, 0)),
            pl.BlockSpec((n_pre, tj), lambda jt, bb: (0, jt)),
            pl.BlockSpec((n_pre, tj), lambda jt, bb: (0, jt)),
            pl.BlockSpec((1, tj), lambda jt, bb: (0, jt)),
        ],
        out_specs=pl.BlockSpec((1, 1, tj), lambda jt, bb: (bb, 0, jt)),
        out_shape=jax.ShapeDtypeStruct((b, 1, n_post), jnp.float32),
        compiler_params=pltpu.CompilerParams(
            dimension_semantics=("parallel", "arbitrary"),
        ),
    )(s_t, input_delays, input_weights, th)
    return out.reshape(b, n_post)


# speedup vs baseline: 6.1225x; 1.2264x over previous
import functools

import jax
import jax.numpy as jnp
from jax.experimental import pallas as pl
from jax.experimental.pallas import tpu as pltpu

_S = 8


def _viota(axis_len, axis):
    shape = [1, 1, 1]
    shape[axis] = axis_len
    return jax.lax.broadcasted_iota(jnp.int32, tuple(shape), axis)


def _bitonic_sort_kv(t, w, n):
    _V = n // _S
    lanes = t.shape[2]
    v_iota = _viota(_V, 0)
    s_iota = _viota(_S, 1)

    def lbit(mask):
        if mask < _V:
            return (v_iota & mask) // mask
        if mask < n:
            sm = mask // _V
            return (s_iota & sm) // sm
        return None

    def sgn(mask):
        b = lbit(mask)
        return None if b is None else (1 - 2 * b).astype(t.dtype)

    u = t * sgn(2)
    m = 2
    while m <= n:
        d = m // 2
        while d >= 1:
            if d <= _V // 2:
                g = _V // (2 * d)
                uv = u.reshape(g, 2, d, _S, lanes)
                wv = w.reshape(g, 2, d, _S, lanes)
                ua, ub = uv[:, 0], uv[:, 1]
                wa, wb = wv[:, 0], wv[:, 1]
                cmp = ua <= ub
                u_lo = jnp.minimum(ua, ub)
                u_hi = jnp.maximum(ua, ub)
                w_lo = jnp.where(cmp, wa, wb)
                w_hi = jnp.where(cmp, wb, wa)
                u = jnp.stack([u_lo, u_hi], axis=1).reshape(_V, _S, lanes)
                w = jnp.stack([w_lo, w_hi], axis=1).reshape(_V, _S, lanes)
            else:
                e = d // _V
                bit_d = (s_iota & e) // e
                first = bit_d == 0
                f = (1 - 2 * bit_d).astype(t.dtype)
                p_u = jnp.where(first, jnp.roll(u, -e, axis=1),
                                jnp.roll(u, e, axis=1))
                p_w = jnp.where(first, jnp.roll(w, -e, axis=1),
                                jnp.roll(w, e, axis=1))
                keep = (f * (u - p_u)) <= 0.0
                u = jnp.where(keep, u, p_u)
                w = jnp.where(keep, w, p_w)
            d //= 2
        if m < n:
            s_m, s_2m = sgn(m), sgn(2 * m)
            u = u * (s_m * s_2m if s_2m is not None else s_m)
        m *= 2
    return u, w


def _cumsum_logical(x):
    _V = x.shape[0]
    v_iota = _viota(_V, 0)
    s_iota = _viota(_S, 1)
    sh = 1
    while sh < _V:
        x = x + jnp.where(v_iota >= sh, jnp.roll(x, sh, axis=0), 0.0)
        sh *= 2
    tot = x[_V - 1:_V]
    sh = 1
    while sh < _S:
        tot = tot + jnp.where(s_iota >= sh, jnp.roll(tot, sh, axis=1), 0.0)
        sh *= 2
    excl = jnp.where(s_iota >= 1, jnp.roll(tot, 1, axis=1), 0.0)
    return x + excl


def _eqtime_kernel(s_ref, d_ref, w_ref, th_ref, o_ref, *, n_pre):
    _V = n_pre // _S
    lanes = d_ref.shape[1]
    t = (s_ref[0] + d_ref[...]).reshape(_V, _S, lanes)
    w = w_ref[...].reshape(_V, _S, lanes)
    t, w = _bitonic_sort_kv(t, w, n_pre)

    cw = _cumsum_logical(w)
    cwt = _cumsum_logical(w * t)

    theta = th_ref[...].reshape(1, 1, lanes)
    eps = 1e-10
    cw_ok = cw > eps
    tmp = (theta + cwt) / jnp.where(cw_ok, cw, 1.0)
    tmp = jnp.where(cw_ok, tmp, jnp.inf)

    v_iota = _viota(_V, 0)
    s_iota = _viota(_S, 1)
    nxt_in_run = jnp.roll(t, -1, axis=0)
    run_head = jnp.roll(jnp.roll(t, -1, axis=1), -1, axis=0)
    run_head = jnp.where(s_iota == _S - 1, jnp.inf, run_head)
    t_next = jnp.where(v_iota == _V - 1, run_head, nxt_in_run)

    valid = (tmp >= t) & (tmp <= t_next)
    cand = jnp.where(valid, tmp, jnp.inf)
    o_ref[0] = jnp.min(cand, axis=(0, 1)).reshape(1, lanes)


def kernel(input_spikes, input_weights, input_delays, thresholds):
    b, n_pre = input_spikes.shape
    n_post = input_weights.shape[1]
    tj = min(128, n_post)
    n_jt = n_post // tj

    s_t = input_spikes.reshape(b, n_pre, 1)
    th = thresholds.reshape(1, n_post)

    grid = (n_jt, b)
    out = pl.pallas_call(
        functools.partial(_eqtime_kernel, n_pre=n_pre),
        grid=grid,
        in_specs=[
            pl.BlockSpec((1, n_pre, 1), lambda jt, bb: (bb, 0, 0)),
            pl.BlockSpec((n_pre, tj), lambda jt, bb: (0, jt)),
            pl.BlockSpec((n_pre, tj), lambda jt, bb: (0, jt)),
            pl.BlockSpec((1, tj), lambda jt, bb: (0, jt)),
        ],
        out_specs=pl.BlockSpec((1, 1, tj), lambda jt, bb: (bb, 0, jt)),
        out_shape=jax.ShapeDtypeStruct((b, 1, n_post), jnp.float32),
        compiler_params=pltpu.CompilerParams(
            dimension_semantics=("parallel", "arbitrary"),
        ),
    )(s_t, input_delays, input_weights, th)
    return out.reshape(b, n_post)


# speedup vs baseline: 9.5647x; 1.5622x over previous
import functools

import jax
import jax.numpy as jnp
from jax.experimental import pallas as pl
from jax.experimental.pallas import tpu as pltpu

_S = 8


def _viota(axis_len, axis):
    shape = [1, 1, 1]
    shape[axis] = axis_len
    return jax.lax.broadcasted_iota(jnp.int32, tuple(shape), axis)


def _sgn_factor(mask, n, v_extent, chunk, base):
    if mask >= n:
        return 1.0
    if mask >= v_extent:
        sm = mask // v_extent
        b = (_viota(_S, 1) & sm) // sm
        return (1 - 2 * b).astype(jnp.float32)
    if mask >= chunk:
        return float(1 - 2 * ((base & mask) // mask))
    civ = jax.lax.broadcasted_iota(jnp.int32, (chunk, 1, 1), 0)
    b = (civ & mask) // mask
    return (1 - 2 * b).astype(jnp.float32)


def _mul_sgn(u, *factors):
    arr = None
    scal = 1.0
    for f in factors:
        if isinstance(f, float):
            scal *= f
        else:
            arr = f if arr is None else arr * f
    if arr is None:
        return u if scal == 1.0 else u * scal
    if scal != 1.0:
        arr = arr * scal
    return u * arr


def _ce_layers_v(u, w, dists, lanes):
    c = u.shape[0]
    for d in dists:
        g = c // (2 * d)
        uv = u.reshape(g, 2, d, _S, lanes)
        wv = w.reshape(g, 2, d, _S, lanes)
        ua, ub = uv[:, 0], uv[:, 1]
        wa, wb = wv[:, 0], wv[:, 1]
        cmp = ua <= ub
        u_lo = jnp.minimum(ua, ub)
        u_hi = jnp.maximum(ua, ub)
        w_lo = jnp.where(cmp, wa, wb)
        w_hi = jnp.where(cmp, wb, wa)
        u = jnp.stack([u_lo, u_hi], axis=1).reshape(c, _S, lanes)
        w = jnp.stack([w_lo, w_hi], axis=1).reshape(c, _S, lanes)
    return u, w


def _ce_layers_s(u, w, es):
    s_iota = _viota(_S, 1)
    for e in es:
        bit = (s_iota & e) // e
        first = bit == 0
        f = (1 - 2 * bit).astype(u.dtype)
        p_u = jnp.where(first, jnp.roll(u, -e, axis=1),
                        jnp.roll(u, e, axis=1))
        p_w = jnp.where(first, jnp.roll(w, -e, axis=1),
                        jnp.roll(w, e, axis=1))
        keep = (f * (u - p_u)) <= 0.0
        u = jnp.where(keep, u, p_u)
        w = jnp.where(keep, w, p_w)
    return u, w


def _cumsum_logical(x):
    _V = x.shape[0]
    v_iota = _viota(_V, 0)
    s_iota = _viota(_S, 1)
    sh = 1
    while sh < _V:
        x = x + jnp.where(v_iota >= sh, jnp.roll(x, sh, axis=0), 0.0)
        sh *= 2
    tot = x[_V - 1:_V]
    sh = 1
    while sh < _S:
        tot = tot + jnp.where(s_iota >= sh, jnp.roll(tot, sh, axis=1), 0.0)
        sh *= 2
    excl = jnp.where(s_iota >= 1, jnp.roll(tot, 1, axis=1), 0.0)
    return x + excl


def _eqtime_kernel(s_ref, d_ref, w_ref, th_ref, o_ref, u_ref, wb_ref,
                   *, n_pre):
    _V = n_pre // _S
    lanes = d_ref.shape[1]
    n = n_pre
    ct = min(16, _V)
    cm = min(8, _V)

    for c in range(_V // ct):
        base = c * ct
        rows = pl.ds(base * _S, ct * _S)
        t_blk = (s_ref[0, rows] + d_ref[rows, :]).reshape(ct, _S, lanes)
        w_blk = w_ref[rows, :].reshape(ct, _S, lanes)
        u_blk = _mul_sgn(t_blk, _sgn_factor(2, n, _V, ct, base))
        m = 2
        while m <= ct and m < n:
            dists = []
            d = m // 2
            while d >= 1:
                dists.append(d)
                d //= 2
            u_blk, w_blk = _ce_layers_v(u_blk, w_blk, dists, lanes)
            u_blk = _mul_sgn(u_blk,
                             _sgn_factor(m, n, _V, ct, base),
                             _sgn_factor(2 * m, n, _V, ct, base))
            m *= 2
        u_ref[pl.ds(base, ct)] = u_blk
        wb_ref[pl.ds(base, ct)] = w_blk

    m = 2 * ct
    while m <= n:
        all_dists = []
        d = m // 2
        while d >= 1:
            all_dists.append(d)
            d //= 2
        s_es = [d // _V for d in all_dists if d >= _V]
        mid = [d for d in all_dists if ct // 2 < d < _V]
        tail = [d for d in all_dists if d <= ct // 2]

        if s_es:
            for c in range(_V // cm):
                sl = pl.ds(c * cm, cm)
                u_blk, w_blk = _ce_layers_s(u_ref[sl], wb_ref[sl], s_es)
                u_ref[sl] = u_blk
                wb_ref[sl] = w_blk
        for d in mid:
            for q in range(_V // (2 * d)):
                for p in range(d // cm):
                    ba = q * 2 * d + p * cm
                    sa, sb = pl.ds(ba, cm), pl.ds(ba + d, cm)
                    ua, ub = u_ref[sa], u_ref[sb]
                    wa, wb = wb_ref[sa], wb_ref[sb]
                    cmp = ua <= ub
                    u_ref[sa] = jnp.minimum(ua, ub)
                    u_ref[sb] = jnp.maximum(ua, ub)
                    wb_ref[sa] = jnp.where(cmp, wa, wb)
                    wb_ref[sb] = jnp.where(cmp, wb, wa)
        for c in range(_V // ct):
            base = c * ct
            sl = pl.ds(base, ct)
            u_blk, w_blk = _ce_layers_v(u_ref[sl], wb_ref[sl], tail, lanes)
            if m < n:
                u_blk = _mul_sgn(u_blk,
                                 _sgn_factor(m, n, _V, ct, base),
                                 _sgn_factor(2 * m, n, _V, ct, base))
            u_ref[sl] = u_blk
            wb_ref[sl] = w_blk
        m *= 2

    t = u_ref[...]
    w = wb_ref[...]
    cw = _cumsum_logical(w)
    cwt = _cumsum_logical(w * t)

    theta = th_ref[...].reshape(1, 1, lanes)
    eps = 1e-10
    cw_ok = cw > eps
    tmp = (theta + cwt) / jnp.where(cw_ok, cw, 1.0)
    tmp = jnp.where(cw_ok, tmp, jnp.inf)

    v_iota = _viota(_V, 0)
    s_iota = _viota(_S, 1)
    nxt_in_run = jnp.roll(t, -1, axis=0)
    run_head = jnp.roll(jnp.roll(t, -1, axis=1), -1, axis=0)
    run_head = jnp.where(s_iota == _S - 1, jnp.inf, run_head)
    t_next = jnp.where(v_iota == _V - 1, run_head, nxt_in_run)

    valid = (tmp >= t) & (tmp <= t_next)
    cand = jnp.where(valid, tmp, jnp.inf)
    o_ref[0] = jnp.min(cand, axis=(0, 1)).reshape(1, lanes)


def kernel(input_spikes, input_weights, input_delays, thresholds):
    b, n_pre = input_spikes.shape
    n_post = input_weights.shape[1]
    tj = min(128, n_post)
    n_jt = n_post // tj

    s_t = input_spikes.reshape(b, n_pre, 1)
    th = thresholds.reshape(1, n_post)

    grid = (n_jt, b)
    out = pl.pallas_call(
        functools.partial(_eqtime_kernel, n_pre=n_pre),
        grid=grid,
        in_specs=[
            pl.BlockSpec((1, n_pre, 1), lambda jt, bb: (bb, 0, 0)),
            pl.BlockSpec((n_pre, tj), lambda jt, bb: (0, jt)),
            pl.BlockSpec((n_pre, tj), lambda jt, bb: (0, jt)),
            pl.BlockSpec((1, tj), lambda jt, bb: (0, jt)),
        ],
        out_specs=pl.BlockSpec((1, 1, tj), lambda jt, bb: (bb, 0, jt)),
        out_shape=jax.ShapeDtypeStruct((b, 1, n_post), jnp.float32),
        scratch_shapes=[
            pltpu.VMEM((n_pre // _S, _S, tj), jnp.float32),
            pltpu.VMEM((n_pre // _S, _S, tj), jnp.float32),
        ],
        compiler_params=pltpu.CompilerParams(
            dimension_semantics=("parallel", "arbitrary"),
        ),
    )(s_t, input_delays, input_weights, th)
    return out.reshape(b, n_post)


# speedup vs baseline: 9.9775x; 1.0432x over previous
import functools

import jax
import jax.numpy as jnp
from jax.experimental import pallas as pl
from jax.experimental.pallas import tpu as pltpu

_S = 8


def _viota(axis_len, axis):
    shape = [1, 1, 1]
    shape[axis] = axis_len
    return jax.lax.broadcasted_iota(jnp.int32, tuple(shape), axis)


def _sgn_factor(mask, n, v_extent, chunk, base):
    if mask >= n:
        return 1.0
    if mask >= v_extent:
        sm = mask // v_extent
        b = (_viota(_S, 1) & sm) // sm
        return (1 - 2 * b).astype(jnp.float32)
    if mask >= chunk:
        return float(1 - 2 * ((base & mask) // mask))
    civ = jax.lax.broadcasted_iota(jnp.int32, (chunk, 1, 1), 0)
    b = (civ & mask) // mask
    return (1 - 2 * b).astype(jnp.float32)


def _mul_sgn(u, *factors):
    arr = None
    scal = 1.0
    for f in factors:
        if isinstance(f, float):
            scal *= f
        else:
            arr = f if arr is None else arr * f
    if arr is None:
        return u if scal == 1.0 else u * scal
    if scal != 1.0:
        arr = arr * scal
    return u * arr


def _ce_layers_v(u, w, dists, lanes):
    c = u.shape[0]
    for d in dists:
        g = c // (2 * d)
        uv = u.reshape(g, 2, d, _S, lanes)
        wv = w.reshape(g, 2, d, _S, lanes)
        ua, ub = uv[:, 0], uv[:, 1]
        wa, wb = wv[:, 0], wv[:, 1]
        cmp = ua <= ub
        u_lo = jnp.minimum(ua, ub)
        u_hi = jnp.maximum(ua, ub)
        w_lo = jnp.where(cmp, wa, wb)
        w_hi = jnp.where(cmp, wb, wa)
        u = jnp.stack([u_lo, u_hi], axis=1).reshape(c, _S, lanes)
        w = jnp.stack([w_lo, w_hi], axis=1).reshape(c, _S, lanes)
    return u, w


def _ce_layers_s(u, w, es):
    s_iota = _viota(_S, 1)
    for e in es:
        bit = (s_iota & e) // e
        first = bit == 0
        f = (1 - 2 * bit).astype(u.dtype)
        p_u = jnp.where(first, jnp.roll(u, -e, axis=1),
                        jnp.roll(u, e, axis=1))
        p_w = jnp.where(first, jnp.roll(w, -e, axis=1),
                        jnp.roll(w, e, axis=1))
        keep = (f * (u - p_u)) <= 0.0
        u = jnp.where(keep, u, p_u)
        w = jnp.where(keep, w, p_w)
    return u, w


def _cumsum_logical(x):
    _V = x.shape[0]
    v_iota = _viota(_V, 0)
    s_iota = _viota(_S, 1)
    sh = 1
    while sh < _V:
        x = x + jnp.where(v_iota >= sh, jnp.roll(x, sh, axis=0), 0.0)
        sh *= 2
    tot = x[_V - 1:_V]
    sh = 1
    while sh < _S:
        tot = tot + jnp.where(s_iota >= sh, jnp.roll(tot, sh, axis=1), 0.0)
        sh *= 2
    excl = jnp.where(s_iota >= 1, jnp.roll(tot, 1, axis=1), 0.0)
    return x + excl


def _eqtime_kernel(s_ref, d_ref, w_ref, th_ref, o_ref, u_ref, wb_ref,
                   ct_ref, *, n_pre):
    _V = n_pre // _S
    lanes = d_ref.shape[1]
    n = n_pre
    ct = min(16, _V)
    cm = min(8, _V)

    for c in range(_V // ct):
        base = c * ct
        rows = pl.ds(base * _S, ct * _S)
        t_blk = (s_ref[0, rows] + d_ref[rows, :]).reshape(ct, _S, lanes)
        w_blk = w_ref[rows, :].reshape(ct, _S, lanes)
        u_blk = _mul_sgn(t_blk, _sgn_factor(2, n, _V, ct, base))
        m = 2
        while m <= ct and m < n:
            dists = []
            d = m // 2
            while d >= 1:
                dists.append(d)
                d //= 2
            u_blk, w_blk = _ce_layers_v(u_blk, w_blk, dists, lanes)
            u_blk = _mul_sgn(u_blk,
                             _sgn_factor(m, n, _V, ct, base),
                             _sgn_factor(2 * m, n, _V, ct, base))
            m *= 2
        u_ref[pl.ds(base, ct)] = u_blk
        wb_ref[pl.ds(base, ct)] = w_blk

    m = 2 * ct
    while m <= n:
        all_dists = []
        d = m // 2
        while d >= 1:
            all_dists.append(d)
            d //= 2
        s_es = [d // _V for d in all_dists if d >= _V]
        mid = [d for d in all_dists if ct // 2 < d < _V]
        tail = [d for d in all_dists if d <= ct // 2]

        if s_es:
            for c in range(_V // cm):
                sl = pl.ds(c * cm, cm)
                u_blk, w_blk = _ce_layers_s(u_ref[sl], wb_ref[sl], s_es)
                u_ref[sl] = u_blk
                wb_ref[sl] = w_blk
        for d in mid:
            for q in range(_V // (2 * d)):
                for p in range(d // cm):
                    ba = q * 2 * d + p * cm
                    sa, sb = pl.ds(ba, cm), pl.ds(ba + d, cm)
                    ua, ub = u_ref[sa], u_ref[sb]
                    wa, wb = wb_ref[sa], wb_ref[sb]
                    cmp = ua <= ub
                    u_ref[sa] = jnp.minimum(ua, ub)
                    u_ref[sb] = jnp.maximum(ua, ub)
                    wb_ref[sa] = jnp.where(cmp, wa, wb)
                    wb_ref[sb] = jnp.where(cmp, wb, wa)
        cv_iota = jax.lax.broadcasted_iota(jnp.int32, (ct, 1, 1), 0)
        for c in range(_V // ct):
            base = c * ct
            sl = pl.ds(base, ct)
            u_blk, w_blk = _ce_layers_v(u_ref[sl], wb_ref[sl], tail, lanes)
            if m < n:
                u_blk = _mul_sgn(u_blk,
                                 _sgn_factor(m, n, _V, ct, base),
                                 _sgn_factor(2 * m, n, _V, ct, base))
                u_ref[sl] = u_blk
                wb_ref[sl] = w_blk
            else:
                cw_blk = w_blk
                cwt_blk = w_blk * u_blk
                sh = 1
                while sh < ct:
                    keepm = cv_iota >= sh
                    cw_blk = cw_blk + jnp.where(
                        keepm, jnp.roll(cw_blk, sh, axis=0), 0.0)
                    cwt_blk = cwt_blk + jnp.where(
                        keepm, jnp.roll(cwt_blk, sh, axis=0), 0.0)
                    sh *= 2
                u_ref[sl] = u_blk
                wb_ref[sl] = cw_blk
                ct_ref[sl] = cwt_blk
        m *= 2

    s_iota = _viota(_S, 1)
    nc = _V // ct
    off_cw = [jnp.zeros((1, _S, lanes), jnp.float32)]
    off_cwt = [jnp.zeros((1, _S, lanes), jnp.float32)]
    for c in range(nc - 1):
        row = pl.ds(c * ct + ct - 1, 1)
        off_cw.append(off_cw[-1] + wb_ref[row])
        off_cwt.append(off_cwt[-1] + ct_ref[row])
    run_tot_cw = off_cw[-1] + wb_ref[pl.ds(_V - 1, 1)]
    run_tot_cwt = off_cwt[-1] + ct_ref[pl.ds(_V - 1, 1)]
    sh = 1
    while sh < _S:
        run_tot_cw = run_tot_cw + jnp.where(
            s_iota >= sh, jnp.roll(run_tot_cw, sh, axis=1), 0.0)
        run_tot_cwt = run_tot_cwt + jnp.where(
            s_iota >= sh, jnp.roll(run_tot_cwt, sh, axis=1), 0.0)
        sh *= 2
    sr_cw = jnp.where(s_iota >= 1, jnp.roll(run_tot_cw, 1, axis=1), 0.0)
    sr_cwt = jnp.where(s_iota >= 1, jnp.roll(run_tot_cwt, 1, axis=1), 0.0)

    theta = th_ref[...].reshape(1, 1, lanes)
    eps = 1e-10
    cv_iota = jax.lax.broadcasted_iota(jnp.int32, (ct, 1, 1), 0)
    run_head = jnp.where(s_iota == _S - 1, jnp.inf,
                         jnp.roll(u_ref[pl.ds(0, 1)], -1, axis=1))
    acc = None
    for c in range(nc):
        sl = pl.ds(c * ct, ct)
        t_blk = u_ref[sl]
        cw = wb_ref[sl] + (off_cw[c] + sr_cw)
        cwt = ct_ref[sl] + (off_cwt[c] + sr_cwt)
        cw_ok = cw > eps
        tmp = (theta + cwt) / jnp.where(cw_ok, cw, 1.0)
        tmp = jnp.where(cw_ok, tmp, jnp.inf)
        nxt = jnp.roll(t_blk, -1, axis=0)
        head = u_ref[pl.ds((c + 1) * ct, 1)] if c < nc - 1 else run_head
        t_next = jnp.where(cv_iota == ct - 1, head, nxt)
        valid = (tmp >= t_blk) & (tmp <= t_next)
        cand = jnp.where(valid, tmp, jnp.inf)
        blk_min = jnp.min(cand, axis=(0, 1)).reshape(1, lanes)
        acc = blk_min if acc is None else jnp.minimum(acc, blk_min)
    o_ref[0] = acc


def kernel(input_spikes, input_weights, input_delays, thresholds):
    b, n_pre = input_spikes.shape
    n_post = input_weights.shape[1]
    tj = min(128, n_post)
    n_jt = n_post // tj

    s_t = input_spikes.reshape(b, n_pre, 1)
    th = thresholds.reshape(1, n_post)

    grid = (n_jt, b)
    out = pl.pallas_call(
        functools.partial(_eqtime_kernel, n_pre=n_pre),
        grid=grid,
        in_specs=[
            pl.BlockSpec((1, n_pre, 1), lambda jt, bb: (bb, 0, 0)),
            pl.BlockSpec((n_pre, tj), lambda jt, bb: (0, jt)),
            pl.BlockSpec((n_pre, tj), lambda jt, bb: (0, jt)),
            pl.BlockSpec((1, tj), lambda jt, bb: (0, jt)),
        ],
        out_specs=pl.BlockSpec((1, 1, tj), lambda jt, bb: (bb, 0, jt)),
        out_shape=jax.ShapeDtypeStruct((b, 1, n_post), jnp.float32),
        scratch_shapes=[
            pltpu.VMEM((n_pre // _S, _S, tj), jnp.float32),
            pltpu.VMEM((n_pre // _S, _S, tj), jnp.float32),
            pltpu.VMEM((n_pre // _S, _S, tj), jnp.float32),
        ],
        compiler_params=pltpu.CompilerParams(
            dimension_semantics=("parallel", "arbitrary"),
        ),
    )(s_t, input_delays, input_weights, th)
    return out.reshape(b, n_post)


# speedup vs baseline: 9.9850x; 1.0008x over previous
import functools

import jax
import jax.numpy as jnp
from jax.experimental import pallas as pl
from jax.experimental.pallas import tpu as pltpu

_S = 8


def _viota(axis_len, axis):
    shape = [1, 1, 1]
    shape[axis] = axis_len
    return jax.lax.broadcasted_iota(jnp.int32, tuple(shape), axis)


def _sgn_factor(mask, n, v_extent, chunk, base):
    if mask >= n:
        return 1.0
    if mask >= v_extent:
        sm = mask // v_extent
        b = (_viota(_S, 1) & sm) // sm
        return (1 - 2 * b).astype(jnp.float32)
    if mask >= chunk:
        return float(1 - 2 * ((base & mask) // mask))
    civ = jax.lax.broadcasted_iota(jnp.int32, (chunk, 1, 1), 0)
    b = (civ & mask) // mask
    return (1 - 2 * b).astype(jnp.float32)


def _mul_sgn(u, *factors):
    arr = None
    scal = 1.0
    for f in factors:
        if isinstance(f, float):
            scal *= f
        else:
            arr = f if arr is None else arr * f
    if arr is None:
        return u if scal == 1.0 else u * scal
    if scal != 1.0:
        arr = arr * scal
    return u * arr


def _ce_layers_v(u, w, dists, lanes):
    c = u.shape[0]
    for d in dists:
        g = c // (2 * d)
        uv = u.reshape(g, 2, d, _S, lanes)
        wv = w.reshape(g, 2, d, _S, lanes)
        ua, ub = uv[:, 0], uv[:, 1]
        wa, wb = wv[:, 0], wv[:, 1]
        cmp = ua <= ub
        u_lo = jnp.minimum(ua, ub)
        u_hi = jnp.maximum(ua, ub)
        w_lo = jnp.where(cmp, wa, wb)
        w_hi = jnp.where(cmp, wb, wa)
        u = jnp.stack([u_lo, u_hi], axis=1).reshape(c, _S, lanes)
        w = jnp.stack([w_lo, w_hi], axis=1).reshape(c, _S, lanes)
    return u, w


def _ce_layers_s(u, w, es):
    s_iota = _viota(_S, 1)
    for e in es:
        bit = (s_iota & e) // e
        first = bit == 0
        f = (1 - 2 * bit).astype(u.dtype)
        p_u = jnp.where(first, jnp.roll(u, -e, axis=1),
                        jnp.roll(u, e, axis=1))
        p_w = jnp.where(first, jnp.roll(w, -e, axis=1),
                        jnp.roll(w, e, axis=1))
        keep = (f * (u - p_u)) <= 0.0
        u = jnp.where(keep, u, p_u)
        w = jnp.where(keep, w, p_w)
    return u, w


def _eqtime_kernel(s_ref, d_ref, w_ref, th_ref, o_ref, u_ref, wb_ref,
                   ct_ref, *, n_pre):
    _V = n_pre // _S
    lanes = d_ref.shape[1]
    n = n_pre
    ct = min(16, _V)
    cm = min(8, _V)

    for c in range(_V // ct):
        base = c * ct
        rows = pl.ds(base * _S, ct * _S)
        t_blk = (s_ref[0, rows] + d_ref[rows, :]).reshape(ct, _S, lanes)
        w_blk = w_ref[rows, :].reshape(ct, _S, lanes)
        u_blk = _mul_sgn(t_blk, _sgn_factor(2, n, _V, ct, base))
        m = 2
        while m <= ct and m < n:
            dists = []
            d = m // 2
            while d >= 1:
                dists.append(d)
                d //= 2
            u_blk, w_blk = _ce_layers_v(u_blk, w_blk, dists, lanes)
            u_blk = _mul_sgn(u_blk,
                             _sgn_factor(m, n, _V, ct, base),
                             _sgn_factor(2 * m, n, _V, ct, base))
            m *= 2
        u_ref[pl.ds(base, ct)] = u_blk
        wb_ref[pl.ds(base, ct)] = w_blk

    m = 2 * ct
    while m <= n:
        all_dists = []
        d = m // 2
        while d >= 1:
            all_dists.append(d)
            d //= 2
        s_es = [d // _V for d in all_dists if d >= _V]
        mid = [d for d in all_dists if ct // 2 < d < _V]
        tail = [d for d in all_dists if d <= ct // 2]

        if s_es:
            for c in range(_V // cm):
                sl = pl.ds(c * cm, cm)
                u_blk, w_blk = _ce_layers_s(u_ref[sl], wb_ref[sl], s_es)
                u_ref[sl] = u_blk
                wb_ref[sl] = w_blk
        for d in mid:
            for q in range(_V // (2 * d)):
                for p in range(d // cm):
                    ba = q * 2 * d + p * cm
                    sa, sb = pl.ds(ba, cm), pl.ds(ba + d, cm)
                    ua, ub = u_ref[sa], u_ref[sb]
                    wa, wb = wb_ref[sa], wb_ref[sb]
                    cmp = ua <= ub
                    u_ref[sa] = jnp.minimum(ua, ub)
                    u_ref[sb] = jnp.maximum(ua, ub)
                    wb_ref[sa] = jnp.where(cmp, wa, wb)
                    wb_ref[sb] = jnp.where(cmp, wb, wa)
        cv_iota = jax.lax.broadcasted_iota(jnp.int32, (ct, 1, 1), 0)
        for c in range(_V // ct):
            base = c * ct
            sl = pl.ds(base, ct)
            u_blk, w_blk = _ce_layers_v(u_ref[sl], wb_ref[sl], tail, lanes)
            if m < n:
                u_blk = _mul_sgn(u_blk,
                                 _sgn_factor(m, n, _V, ct, base),
                                 _sgn_factor(2 * m, n, _V, ct, base))
                u_ref[sl] = u_blk
                wb_ref[sl] = w_blk
            else:
                cw_blk = w_blk
                cwt_blk = w_blk * u_blk
                sh = 1
                while sh < ct:
                    keepm = cv_iota >= sh
                    cw_blk = cw_blk + jnp.where(
                        keepm, jnp.roll(cw_blk, sh, axis=0), 0.0)
                    cwt_blk = cwt_blk + jnp.where(
                        keepm, jnp.roll(cwt_blk, sh, axis=0), 0.0)
                    sh *= 2
                u_ref[sl] = u_blk
                wb_ref[sl] = cw_blk
                ct_ref[sl] = cwt_blk
        m *= 2

    s_iota = _viota(_S, 1)
    nc = _V // ct
    off_cw = [jnp.zeros((1, _S, lanes), jnp.float32)]
    off_cwt = [jnp.zeros((1, _S, lanes), jnp.float32)]
    for c in range(nc - 1):
        row = pl.ds(c * ct + ct - 1, 1)
        off_cw.append(off_cw[-1] + wb_ref[row])
        off_cwt.append(off_cwt[-1] + ct_ref[row])
    run_tot_cw = off_cw[-1] + wb_ref[pl.ds(_V - 1, 1)]
    run_tot_cwt = off_cwt[-1] + ct_ref[pl.ds(_V - 1, 1)]
    sh = 1
    while sh < _S:
        run_tot_cw = run_tot_cw + jnp.where(
            s_iota >= sh, jnp.roll(run_tot_cw, sh, axis=1), 0.0)
        run_tot_cwt = run_tot_cwt + jnp.where(
            s_iota >= sh, jnp.roll(run_tot_cwt, sh, axis=1), 0.0)
        sh *= 2
    sr_cw = jnp.where(s_iota >= 1, jnp.roll(run_tot_cw, 1, axis=1), 0.0)
    sr_cwt = jnp.where(s_iota >= 1, jnp.roll(run_tot_cwt, 1, axis=1), 0.0)

    theta = th_ref[...].reshape(1, 1, lanes)
    eps = 1e-10
    cv_iota = jax.lax.broadcasted_iota(jnp.int32, (ct, 1, 1), 0)
    run_head = jnp.where(s_iota == _S - 1, jnp.inf,
                         jnp.roll(u_ref[pl.ds(0, 1)], -1, axis=1))
    acc = None
    for c in range(nc):
        sl = pl.ds(c * ct, ct)
        t_blk = u_ref[sl]
        cw = wb_ref[sl] + (off_cw[c] + sr_cw)
        cwt = ct_ref[sl] + (off_cwt[c] + sr_cwt)
        cw_ok = cw > eps
        tmp = (theta + cwt) / jnp.where(cw_ok, cw, 1.0)
        tmp = jnp.where(cw_ok, tmp, jnp.inf)
        nxt = jnp.roll(t_blk, -1, axis=0)
        head = u_ref[pl.ds((c + 1) * ct, 1)] if c < nc - 1 else run_head
        t_next = jnp.where(cv_iota == ct - 1, head, nxt)
        valid = (tmp >= t_blk) & (tmp <= t_next)
        cand = jnp.where(valid, tmp, jnp.inf)
        blk_min = jnp.min(cand, axis=(0, 1)).reshape(1, lanes)
        acc = blk_min if acc is None else jnp.minimum(acc, blk_min)
    o_ref[0] = acc


def kernel(input_spikes, input_weights, input_delays, thresholds):
    b, n_pre = input_spikes.shape
    n_post = input_weights.shape[1]
    tj = min(128, n_post)
    n_jt = n_post // tj

    s_t = input_spikes.reshape(b, n_pre, 1)
    th = thresholds.reshape(1, n_post)

    grid = (n_jt, b)
    out = pl.pallas_call(
        functools.partial(_eqtime_kernel, n_pre=n_pre),
        grid=grid,
        in_specs=[
            pl.BlockSpec((1, n_pre, 1), lambda jt, bb: (bb, 0, 0)),
            pl.BlockSpec((n_pre, tj), lambda jt, bb: (0, jt)),
            pl.BlockSpec((n_pre, tj), lambda jt, bb: (0, jt)),
            pl.BlockSpec((1, tj), lambda jt, bb: (0, jt)),
        ],
        out_specs=pl.BlockSpec((1, 1, tj), lambda jt, bb: (bb, 0, jt)),
        out_shape=jax.ShapeDtypeStruct((b, 1, n_post), jnp.float32),
        scratch_shapes=[
            pltpu.VMEM((n_pre // _S, _S, tj), jnp.float32),
            pltpu.VMEM((n_pre // _S, _S, tj), jnp.float32),
            pltpu.VMEM((n_pre // _S, _S, tj), jnp.float32),
        ],
        compiler_params=pltpu.CompilerParams(
            dimension_semantics=("parallel", "arbitrary"),
        ),
    )(s_t, input_delays, input_weights, th)
    return out.reshape(b, n_post)
